# baseline (speedup 1.0000x reference)
import jax
import jax.numpy as jnp
from jax import lax
from jax.experimental import pallas as pl
from jax.experimental.pallas import tpu as pltpu
from jax.experimental.pallas import tpu_sc as plsc

B = 1024
L = 200
NUM_CLASSES = 100000
NGROUP = 25
WPR = NGROUP * 128
NC = 2
NS = 16
NW = NC * NS
ROWS_PER_W = B // NW
LPAD = 208
NVEC = LPAD // 16
NBUF = 2
SLAB = 8
SLABS_PER_W = ROWS_PER_W // SLAB
SLAB_WORDS = SLAB * WPR


def _sc_body(idx_hbm, out_hbm, idx_v, stage, sem):
    wid = lax.axis_index("s") * NC + lax.axis_index("c")
    base = wid * ROWS_PER_W

    pltpu.sync_copy(idx_hbm.at[pl.ds(base, ROWS_PER_W)], idx_v)

    zero16 = jnp.zeros((16,), jnp.int32)
    ones_m = jnp.ones((16,), jnp.bool_)

    def init_zero(i, c):
        stage[pl.ds(i * 16, 16)] = zero16
        return c

    lax.fori_loop(0, NBUF * SLAB_WORDS // 16, init_zero, 0)

    def bitmap_pos(c_idx, boff, r8):
        w = boff + r8 * WPR + ((c_idx >> 12) << 7) + ((c_idx >> 2) & 127)
        bit = jnp.int32(1) << (((c_idx & 3) << 3) + ((c_idx >> 9) & 7))
        return w, bit

    def per_slab(s, c):
        b = lax.rem(s, NBUF)
        boff = b * SLAB_WORDS

        @pl.when(s >= NBUF)
        def _():
            pltpu.make_async_copy(
                stage.at[pl.ds(boff, SLAB_WORDS)],
                out_hbm.at[pl.ds(base * WPR + s * SLAB_WORDS, SLAB_WORDS)],
                sem,
            ).wait()

            def rezero_row(r8, c2):
                lr = (s - NBUF) * SLAB + r8
                for k in range(NVEC):
                    iv = idx_v[lr, pl.ds(k * 16, 16)]
                    w, _ = bitmap_pos(iv, boff, r8)
                    plsc.store_scatter(stage, [w], zero16)
                return c2

            lax.fori_loop(0, SLAB, rezero_row, 0)

        def scatter_row(r8, c2):
            lr = s * SLAB + r8
            for k in range(NVEC):
                iv = idx_v[lr, pl.ds(k * 16, 16)]
                w, bit = bitmap_pos(iv, boff, r8)

                def cond(carry):
                    return jnp.any(carry)

                def body(m):
                    old = plsc.load_gather(stage, [w])
                    plsc.store_scatter(stage, [w], old | bit, mask=m)
                    back = plsc.load_gather(stage, [w])
                    return jnp.logical_and(m, (back & bit) != bit)

                lax.while_loop(cond, body, ones_m)
            return c2

        lax.fori_loop(0, SLAB, scatter_row, 0)

        pltpu.make_async_copy(
            stage.at[pl.ds(boff, SLAB_WORDS)],
            out_hbm.at[pl.ds(base * WPR + s * SLAB_WORDS, SLAB_WORDS)],
            sem,
        ).start()
        return c

    lax.fori_loop(0, SLABS_PER_W, per_slab, 0)

    def drain(k, c):
        pltpu.make_async_copy(
            stage.at[pl.ds(lax.rem(k, NBUF) * SLAB_WORDS, SLAB_WORDS)],
            out_hbm.at[pl.ds(base * WPR + k * SLAB_WORDS, SLAB_WORDS)],
            sem,
        ).wait()
        return c

    lax.fori_loop(0, NBUF, drain, 0)


def _tc_body(bm_ref, out_ref):
    xt = bm_ref[...]
    ow = out_ref.bitcast(jnp.int32)
    lanes = jnp.int32(0x01010101)
    for j in range(8):
        ow[128 * j:128 * (j + 1), :] = (xt >> j) & lanes


def kernel(indices):
    indices = indices.astype(jnp.int32)
    pad = jnp.broadcast_to(indices[:, -1:], (B, LPAD - L))
    idx2 = jnp.concatenate([indices, pad], axis=1)

    mesh = plsc.VectorSubcoreMesh(core_axis_name="c", subcore_axis_name="s")
    sc = pl.kernel(
        _sc_body,
        out_type=jax.ShapeDtypeStruct((B * WPR,), jnp.int32),
        mesh=mesh,
        scratch_types=[
            pltpu.VMEM((ROWS_PER_W, LPAD), jnp.int32),
            pltpu.VMEM((NBUF * SLAB_WORDS,), jnp.int32),
            pltpu.SemaphoreType.DMA,
        ],
        compiler_params=pltpu.CompilerParams(needs_layout_passes=False),
    )
    bitmap_t = sc(idx2).reshape(B, WPR).T

    out_t8 = pl.pallas_call(
        _tc_body,
        out_shape=jax.ShapeDtypeStruct((NUM_CLASSES, B), jnp.uint8),
        grid=(B // 128, NGROUP),
        in_specs=[pl.BlockSpec((128, 128), lambda rb, g: (g, rb))],
        out_specs=pl.BlockSpec((4096, 128), lambda rb, g: (g, rb)),
    )(bitmap_t)
    return (out_t8 != 0).T


# speedup vs baseline: 1.4185x; 1.4185x over previous
import jax
import jax.numpy as jnp
from jax import lax
from jax.experimental import pallas as pl
from jax.experimental.pallas import tpu as pltpu
from jax.experimental.pallas import tpu_sc as plsc

B = 1024
L = 200
NUM_CLASSES = 100000
NGROUP = 25
WPR = NGROUP * 128
NC = 2
NS = 16
NW = NC * NS
ROWS_PER_W = B // NW
LPAD = 208
NVEC = LPAD // 16
NBUF = 2
SLAB = 8
SLABS_PER_W = ROWS_PER_W // SLAB
SLAB_WORDS = SLAB * WPR


def _sc_body(idx_hbm, out_hbm, idx_v, stage, sem):
    wid = lax.axis_index("s") * NC + lax.axis_index("c")
    base = wid * ROWS_PER_W

    pltpu.sync_copy(idx_hbm.at[pl.ds(base, ROWS_PER_W)], idx_v)

    zero16 = jnp.zeros((16,), jnp.int32)
    ones_m = jnp.ones((16,), jnp.bool_)

    def init_zero(i, c):
        stage[pl.ds(i * 16, 16)] = zero16
        return c

    lax.fori_loop(0, NBUF * SLAB_WORDS // 16, init_zero, 0)

    def bitmap_pos(c_idx, boff, r8):
        w = boff + r8 * WPR + ((c_idx >> 12) << 7) + ((c_idx >> 2) & 127)
        bit = jnp.int32(1) << (((c_idx & 3) << 3) + ((c_idx >> 9) & 7))
        return w, bit

    def per_slab(s, c):
        b = lax.rem(s, NBUF)
        boff = b * SLAB_WORDS

        @pl.when(s >= NBUF)
        def _():
            pltpu.make_async_copy(
                stage.at[pl.ds(boff, SLAB_WORDS)],
                out_hbm.at[pl.ds(base * WPR + s * SLAB_WORDS, SLAB_WORDS)],
                sem,
            ).wait()

            def rezero_row(r8, c2):
                lr = (s - NBUF) * SLAB + r8
                for k in range(NVEC):
                    iv = idx_v[lr, pl.ds(k * 16, 16)]
                    w, _ = bitmap_pos(iv, boff, r8)
                    plsc.store_scatter(stage, [w], zero16)
                return c2

            lax.fori_loop(0, SLAB, rezero_row, 0)

        def scatter_row(r8, c2):
            lr = s * SLAB + r8
            for k in range(NVEC):
                iv = idx_v[lr, pl.ds(k * 16, 16)]
                w, bit = bitmap_pos(iv, boff, r8)

                def cond(carry):
                    return jnp.any(carry)

                def body(m):
                    old = plsc.load_gather(stage, [w])
                    plsc.store_scatter(stage, [w], old | bit, mask=m)
                    back = plsc.load_gather(stage, [w])
                    return jnp.logical_and(m, (back & bit) != bit)

                lax.while_loop(cond, body, ones_m)
            return c2

        lax.fori_loop(0, SLAB, scatter_row, 0)

        pltpu.make_async_copy(
            stage.at[pl.ds(boff, SLAB_WORDS)],
            out_hbm.at[pl.ds(base * WPR + s * SLAB_WORDS, SLAB_WORDS)],
            sem,
        ).start()
        return c

    lax.fori_loop(0, SLABS_PER_W, per_slab, 0)

    def drain(k, c):
        pltpu.make_async_copy(
            stage.at[pl.ds(lax.rem(k, NBUF) * SLAB_WORDS, SLAB_WORDS)],
            out_hbm.at[pl.ds(base * WPR + k * SLAB_WORDS, SLAB_WORDS)],
            sem,
        ).wait()
        return c

    lax.fori_loop(0, NBUF, drain, 0)


def _tc_body(bm_ref, out_ref):
    xt = bm_ref[...]
    ow = out_ref.bitcast(jnp.int32)
    lanes = jnp.int32(0x01010101)
    for j in range(8):
        ow[128 * j:128 * (j + 1), :] = (xt >> j) & lanes


def kernel(indices):
    indices = indices.astype(jnp.int32)
    pad = jnp.broadcast_to(indices[:, -1:], (B, LPAD - L))
    idx2 = jnp.concatenate([indices, pad], axis=1)

    mesh = plsc.VectorSubcoreMesh(core_axis_name="c", subcore_axis_name="s")
    sc = pl.kernel(
        _sc_body,
        out_type=jax.ShapeDtypeStruct((B * WPR,), jnp.int32),
        mesh=mesh,
        scratch_types=[
            pltpu.VMEM((ROWS_PER_W, LPAD), jnp.int32),
            pltpu.VMEM((NBUF * SLAB_WORDS,), jnp.int32),
            pltpu.SemaphoreType.DMA,
        ],
        compiler_params=pltpu.CompilerParams(needs_layout_passes=False),
    )
    bitmap_t = sc(idx2).reshape(B, WPR).T

    out_t8 = pl.pallas_call(
        _tc_body,
        out_shape=jax.ShapeDtypeStruct((NUM_CLASSES, B), jnp.uint8),
        grid=(NGROUP,),
        in_specs=[pl.BlockSpec((128, B), lambda g: (g, 0))],
        out_specs=pl.BlockSpec((4096, B), lambda g: (g, 0)),
    )(bitmap_t)
    return (out_t8 != 0).T
